# baseline (speedup 1.0000x reference)
import functools
import jax
import jax.numpy as jnp
from jax import lax
from jax.experimental import pallas as pl
from jax.experimental.pallas import tpu as pltpu
from jax.experimental.pallas import tpu_sc as plsc

NGRAPH = 32
NW = 32
NSUB = 16
CHUNK = 64
LANES = 16

_MESH = plsc.VectorSubcoreMesh(core_axis_name="c", subcore_axis_name="s")


def _round_up(a, m):
    return (a + m - 1) // m * m


def _sc_degree(cidx, npad, nch):
    stripe = npad // NSUB

    def body(cidx_hbm, deg_hbm, cidx_v, ones_v, zb_v, acc):
        c = lax.axis_index("c")
        s = lax.axis_index("s")
        wid = s * 2 + c
        for i in range(stripe // LANES):
            zb_v[pl.ds(i * LANES, LANES)] = jnp.zeros((LANES,), jnp.float32)
        for i in range(CHUNK // LANES):
            ones_v[pl.ds(i * LANES, LANES)] = jnp.ones((LANES,), jnp.float32)
        pltpu.sync_copy(zb_v, acc.at[pl.ds(s * stripe, stripe)])
        pltpu.sync_copy(cidx_hbm.at[wid], cidx_v)
        plsc.subcore_barrier()

        def chunk(k, carry):
            pltpu.sync_copy(ones_v, acc.at[cidx_v.at[k]], add=True)
            return carry

        lax.fori_loop(0, nch, chunk, 0)
        plsc.subcore_barrier()
        pltpu.sync_copy(acc.at[pl.ds(s * stripe, stripe)],
                        deg_hbm.at[c, pl.ds(s * stripe, stripe)])

    fn = pl.kernel(
        body,
        out_type=jax.ShapeDtypeStruct((2, npad), jnp.float32),
        mesh=_MESH,
        scratch_types=[
            pltpu.VMEM((nch, CHUNK), jnp.int32),
            pltpu.VMEM((CHUNK,), jnp.float32),
            pltpu.VMEM((stripe,), jnp.float32),
            pltpu.VMEM_SHARED((npad,), jnp.float32),
        ],
    )
    return fn(cidx)


def _sc_agg(y, idx2, npad, nch, d):
    stripe = npad // NSUB

    nslot = 4

    def body(y_hbm, idx_hbm, out_hbm,
             ib0, ib1, ib2, ib3, ixr0, ixr1, ixr2, ixr3,
             ixc0, ixc1, ixc2, ixc3, buf0, buf1, buf2, buf3, acc, *sem):
        c = lax.axis_index("c")
        s = lax.axis_index("s")
        wid = s * 2 + c
        ibs = (ib0, ib1, ib2, ib3)
        ixrs = (ixr0, ixr1, ixr2, ixr3)
        ixcs = (ixc0, ixc1, ixc2, ixc3)
        bufs = (buf0, buf1, buf2, buf3)
        semi, semg, sems = sem[:nslot], sem[nslot:2 * nslot], sem[2 * nslot:]

        for b in range(nslot):
            pltpu.async_copy(idx_hbm.at[wid, b], ibs[b], semi[b])

        zr = CHUNK
        def zrow(i, carry):
            for j in range(d // LANES):
                buf0[i, pl.ds(j * LANES, LANES)] = jnp.zeros((LANES,), jnp.float32)
            return carry

        lax.fori_loop(0, zr, zrow, 0)

        def zissue(i, carry):
            pltpu.async_copy(buf0.at[pl.ds(0, zr)],
                             acc.at[pl.ds(s * stripe + i * zr, zr)], sem[-1])
            return carry

        def zdrain(i, carry):
            pltpu.make_async_copy(
                buf0.at[pl.ds(0, zr)],
                acc.at[pl.ds(s * stripe + i * zr, zr)], sem[-1]).wait()
            return carry

        lax.fori_loop(0, stripe // zr, zissue, 0)
        lax.fori_loop(0, stripe // zr, zdrain, 0)
        plsc.subcore_barrier()

        def unpack(ib, ixr, ixc):
            for j in range(CHUNK // LANES):
                w = ib[0, pl.ds(j * LANES, LANES)]
                ixr[pl.ds(j * LANES, LANES)] = w & 0xFFFF
                ixc[pl.ds(j * LANES, LANES)] = lax.shift_right_logical(w, 16)

        def chunk(i, carry):
            for b in range(nslot):
                kb = nslot * i + b

                @pl.when(kb >= nslot)
                def _(b=b):
                    pltpu.make_async_copy(
                        bufs[b], acc.at[ixcs[b]], sems[b]).wait()

                pltpu.make_async_copy(
                    idx_hbm.at[wid, kb], ibs[b], semi[b]).wait()
                unpack(ibs[b], ixrs[b], ixcs[b])

                @pl.when(kb + nslot < nch)
                def _(b=b, kb=kb):
                    pltpu.async_copy(
                        idx_hbm.at[wid, kb + nslot], ibs[b], semi[b])

                pltpu.async_copy(y_hbm.at[ixrs[b]], bufs[b], semg[b])

            for b in range(nslot):
                pltpu.make_async_copy(
                    y_hbm.at[ixrs[b]], bufs[b], semg[b]).wait()
                pltpu.async_copy(
                    bufs[b], acc.at[ixcs[b]], sems[b], add=True)
            return carry

        lax.fori_loop(0, nch // nslot, chunk, 0)
        for b in range(nslot):
            pltpu.make_async_copy(bufs[b], acc.at[ixcs[b]],
                                  sems[b]).wait()
        plsc.subcore_barrier()
        pltpu.sync_copy(acc.at[pl.ds(s * stripe, stripe)],
                        out_hbm.at[c, pl.ds(s * stripe, stripe)])

    fn = pl.kernel(
        body,
        out_type=jax.ShapeDtypeStruct((2, npad, d), jnp.float32),
        mesh=_MESH,
        scratch_types=(
            [pltpu.VMEM((1, CHUNK), jnp.int32)] * nslot
            + [pltpu.VMEM((CHUNK,), jnp.int32)] * (2 * nslot)
            + [pltpu.VMEM((CHUNK, d), jnp.float32)] * nslot
            + [pltpu.VMEM_SHARED((npad, d), jnp.float32)]
            + [pltpu.SemaphoreType.DMA] * (3 * nslot + 1)
        ),
    )
    return fn(y, idx2)


def _tc_prep_body(degp_ref, x_ref, w_ref, y_ref, dis_ref):
    deg = degp_ref[0] + degp_ref[1] + 1.0
    dis = lax.rsqrt(deg)
    xw = jnp.dot(x_ref[...], w_ref[...], preferred_element_type=jnp.float32)
    y_ref[...] = dis * xw
    dis_ref[...] = dis


def _tc_prep(degp, x, w, npad, d, blk):
    grid = (npad // blk,)
    return pl.pallas_call(
        _tc_prep_body,
        grid=grid,
        in_specs=[
            pl.BlockSpec((2, blk, 1), lambda i: (0, i, 0)),
            pl.BlockSpec((blk, d), lambda i: (i, 0)),
            pl.BlockSpec((d, d), lambda i: (0, 0)),
        ],
        out_specs=[
            pl.BlockSpec((blk, d), lambda i: (i, 0)),
            pl.BlockSpec((blk, 1), lambda i: (i, 0)),
        ],
        out_shape=[
            jax.ShapeDtypeStruct((npad, d), jnp.float32),
            jax.ShapeDtypeStruct((npad, 1), jnp.float32),
        ],
    )(degp, x, w)


def _tc_mid_body(dis_ref, p_ref, y_ref, w_ref, b_ref, y2_ref):
    dis = dis_ref[...]
    h = jnp.maximum(dis * (p_ref[0] + p_ref[1] + y_ref[...]) + b_ref[...], 0.0)
    y2_ref[...] = dis * jnp.dot(h, w_ref[...], preferred_element_type=jnp.float32)


def _tc_mid(dis, p, y, w, b, npad, d, blk):
    grid = (npad // blk,)
    return pl.pallas_call(
        _tc_mid_body,
        grid=grid,
        in_specs=[
            pl.BlockSpec((blk, 1), lambda i: (i, 0)),
            pl.BlockSpec((2, blk, d), lambda i: (0, i, 0)),
            pl.BlockSpec((blk, d), lambda i: (i, 0)),
            pl.BlockSpec((d, d), lambda i: (0, 0)),
            pl.BlockSpec((1, d), lambda i: (0, 0)),
        ],
        out_specs=pl.BlockSpec((blk, d), lambda i: (i, 0)),
        out_shape=jax.ShapeDtypeStruct((npad, d), jnp.float32),
    )(dis, p, y, w, b)


def _tc_final_body(dis_ref, p_ref, y_ref, b_ref, batch_ref, wfc_ref, bfc_ref,
                   out_ref, pooled, cnt, *, blk, nblocks, n):
    i = pl.program_id(0)

    @pl.when(i == 0)
    def _():
        pooled[...] = jnp.zeros_like(pooled)
        cnt[...] = jnp.zeros_like(cnt)

    dis = dis_ref[...]
    h = jnp.maximum(dis * (p_ref[0] + p_ref[1] + y_ref[...]) + b_ref[...], 0.0)
    valid = (i * blk + lax.broadcasted_iota(jnp.int32, (blk, 1), 0)) < n
    h = jnp.where(valid, h, 0.0)
    labels = lax.broadcasted_iota(jnp.int32, (blk, NGRAPH), 1)
    onehot = jnp.where(valid, (batch_ref[...] == labels).astype(jnp.float32),
                       0.0)
    pooled[...] += lax.dot_general(
        onehot, h, (((0,), (0,)), ((), ())), preferred_element_type=jnp.float32)
    cnt[...] += jnp.sum(onehot, axis=0, keepdims=True)

    @pl.when(i == nblocks - 1)
    def _():
        g = pooled[...] / jnp.maximum(cnt[...], 1.0).T
        out_ref[...] = jnp.dot(g, wfc_ref[...],
                               preferred_element_type=jnp.float32) + bfc_ref[...]


def _tc_final(dis, p, y, b, batch2d, wfc, bfc, npad, d, o, blk, n):
    nblocks = npad // blk
    body = functools.partial(_tc_final_body, blk=blk, nblocks=nblocks, n=n)
    return pl.pallas_call(
        body,
        grid=(nblocks,),
        in_specs=[
            pl.BlockSpec((blk, 1), lambda i: (i, 0)),
            pl.BlockSpec((2, blk, d), lambda i: (0, i, 0)),
            pl.BlockSpec((blk, d), lambda i: (i, 0)),
            pl.BlockSpec((1, d), lambda i: (0, 0)),
            pl.BlockSpec((blk, 1), lambda i: (i, 0)),
            pl.BlockSpec((d, o), lambda i: (0, 0)),
            pl.BlockSpec((1, o), lambda i: (0, 0)),
        ],
        out_specs=pl.BlockSpec((NGRAPH, o), lambda i: (0, 0)),
        out_shape=jax.ShapeDtypeStruct((NGRAPH, o), jnp.float32),
        scratch_shapes=[
            pltpu.VMEM((NGRAPH, d), jnp.float32),
            pltpu.VMEM((1, NGRAPH), jnp.float32),
        ],
    )(dis, p, y, b, batch2d, wfc, bfc)


@jax.jit
def kernel(x, edge_index, batch, W1, b1, W2, b2, Wfc, bfc):
    n, d = x.shape
    e = edge_index.shape[1]
    o = Wfc.shape[1]
    blk = 1024

    npad = _round_up(n + 1, max(blk, NSUB * CHUNK))
    epad = _round_up(e, NW * CHUNK * 4)
    nch = epad // (NW * CHUNK)

    batch_p = jnp.concatenate(
        [batch, jnp.full((npad - n,), NGRAPH, batch.dtype)]).reshape(npad, 1)
    pad_nodes = n + jnp.arange(epad - e, dtype=edge_index.dtype) % (npad - n)
    row_p = jnp.concatenate([edge_index[0], pad_nodes])
    col_p = jnp.concatenate([edge_index[1], pad_nodes])
    idx2 = (row_p | (col_p << 16)).reshape(NW, nch, 1, CHUNK)
    cidx = col_p.reshape(NW, nch, CHUNK)

    degp = _sc_degree(cidx, npad, nch).reshape(2, npad, 1)
    y1, dis = _tc_prep(degp, x, W1, npad, d, blk)
    p1 = _sc_agg(y1, idx2, npad, nch, d)
    y2 = _tc_mid(dis, p1, y1, W2, b1.reshape(1, d), npad, d, blk)
    p2 = _sc_agg(y2, idx2, npad, nch, d)
    return _tc_final(dis, p2, y2, b2.reshape(1, d), batch_p, Wfc,
                     bfc.reshape(1, o), npad, d, o, blk, n)


# speedup vs baseline: 1.0004x; 1.0004x over previous
import functools
import jax
import jax.numpy as jnp
from jax import lax
from jax.experimental import pallas as pl
from jax.experimental.pallas import tpu as pltpu
from jax.experimental.pallas import tpu_sc as plsc

NGRAPH = 32
NW = 32
NSUB = 16
CHUNK = 64
LANES = 16

_MESH = plsc.VectorSubcoreMesh(core_axis_name="c", subcore_axis_name="s")


def _round_up(a, m):
    return (a + m - 1) // m * m


def _sc_degree(cidx, npad, nch):
    stripe = npad // NSUB

    def body(cidx_hbm, deg_hbm, cidx_v, ones_v, zb_v, acc):
        c = lax.axis_index("c")
        s = lax.axis_index("s")
        wid = s * 2 + c
        for i in range(stripe // LANES):
            zb_v[pl.ds(i * LANES, LANES)] = jnp.zeros((LANES,), jnp.float32)
        for i in range(CHUNK // LANES):
            ones_v[pl.ds(i * LANES, LANES)] = jnp.ones((LANES,), jnp.float32)
        pltpu.sync_copy(zb_v, acc.at[pl.ds(s * stripe, stripe)])
        pltpu.sync_copy(cidx_hbm.at[wid], cidx_v)
        plsc.subcore_barrier()

        def chunk(k, carry):
            pltpu.sync_copy(ones_v, acc.at[cidx_v.at[k]], add=True)
            return carry

        lax.fori_loop(0, nch, chunk, 0)
        plsc.subcore_barrier()
        pltpu.sync_copy(acc.at[pl.ds(s * stripe, stripe)],
                        deg_hbm.at[c, pl.ds(s * stripe, stripe)])

    fn = pl.kernel(
        body,
        out_type=jax.ShapeDtypeStruct((2, npad), jnp.float32),
        mesh=_MESH,
        scratch_types=[
            pltpu.VMEM((nch, CHUNK), jnp.int32),
            pltpu.VMEM((CHUNK,), jnp.float32),
            pltpu.VMEM((stripe,), jnp.float32),
            pltpu.VMEM_SHARED((npad,), jnp.float32),
        ],
    )
    return fn(cidx)


def _sc_agg(y, idx2, npad, nch, d):
    stripe = npad // NSUB

    nslot = 4

    def body(y_hbm, idx_hbm, out_hbm,
             ib0, ib1, ib2, ib3, ixr0, ixr1, ixr2, ixr3,
             ixc0, ixc1, ixc2, ixc3, buf0, buf1, buf2, buf3, acc, *sem):
        c = lax.axis_index("c")
        s = lax.axis_index("s")
        wid = s * 2 + c
        ibs = (ib0, ib1, ib2, ib3)
        ixrs = (ixr0, ixr1, ixr2, ixr3)
        ixcs = (ixc0, ixc1, ixc2, ixc3)
        bufs = (buf0, buf1, buf2, buf3)
        semi, semg, sems = sem[:nslot], sem[nslot:2 * nslot], sem[2 * nslot:]

        for b in range(nslot):
            pltpu.async_copy(idx_hbm.at[wid, b], ibs[b], semi[b])

        zr = CHUNK
        def zrow(i, carry):
            for j in range(d // LANES):
                buf0[i, pl.ds(j * LANES, LANES)] = jnp.zeros((LANES,), jnp.float32)
            return carry

        lax.fori_loop(0, zr, zrow, 0)

        def zissue(i, carry):
            pltpu.async_copy(buf0.at[pl.ds(0, zr)],
                             acc.at[pl.ds(s * stripe + i * zr, zr)], sem[-1])
            return carry

        def zdrain(i, carry):
            pltpu.make_async_copy(
                buf0.at[pl.ds(0, zr)],
                acc.at[pl.ds(s * stripe + i * zr, zr)], sem[-1]).wait()
            return carry

        lax.fori_loop(0, stripe // zr, zissue, 0)
        lax.fori_loop(0, stripe // zr, zdrain, 0)
        plsc.subcore_barrier()

        def unpack(ib, ixr, ixc):
            for j in range(CHUNK // LANES):
                w = ib[0, pl.ds(j * LANES, LANES)]
                ixr[pl.ds(j * LANES, LANES)] = w & 0xFFFF
                ixc[pl.ds(j * LANES, LANES)] = lax.shift_right_logical(w, 16)

        def chunk(i, carry):
            for b in range(nslot):
                kb = nslot * i + b

                @pl.when(kb >= nslot)
                def _(b=b):
                    pltpu.make_async_copy(
                        bufs[b], acc.at[ixcs[b]], sems[b]).wait()

                pltpu.make_async_copy(
                    idx_hbm.at[wid, kb], ibs[b], semi[b]).wait()
                unpack(ibs[b], ixrs[b], ixcs[b])

                @pl.when(kb + nslot < nch)
                def _(b=b, kb=kb):
                    pltpu.async_copy(
                        idx_hbm.at[wid, kb + nslot], ibs[b], semi[b])

                pltpu.async_copy(y_hbm.at[ixrs[b]], bufs[b], semg[b])

            for b in range(nslot):
                pltpu.make_async_copy(
                    y_hbm.at[ixrs[b]], bufs[b], semg[b]).wait()
                pltpu.async_copy(
                    bufs[b], acc.at[ixcs[b]], sems[b], add=True)
            return carry

        lax.fori_loop(0, nch // nslot, chunk, 0)
        for b in range(nslot):
            pltpu.make_async_copy(bufs[b], acc.at[ixcs[b]],
                                  sems[b]).wait()
        plsc.subcore_barrier()
        pltpu.sync_copy(acc.at[pl.ds(s * stripe, stripe)],
                        out_hbm.at[c, pl.ds(s * stripe, stripe)])

    fn = pl.kernel(
        body,
        out_type=jax.ShapeDtypeStruct((2, npad, d), jnp.float32),
        mesh=_MESH,
        scratch_types=(
            [pltpu.VMEM((1, CHUNK), jnp.int32)] * nslot
            + [pltpu.VMEM((CHUNK,), jnp.int32)] * (2 * nslot)
            + [pltpu.VMEM((CHUNK, d), jnp.float32)] * nslot
            + [pltpu.VMEM_SHARED((npad, d), jnp.float32)]
            + [pltpu.SemaphoreType.DMA] * (3 * nslot + 1)
        ),
    )
    return fn(y, idx2)


def _tc_prep_body(degp_ref, x_ref, w_ref, y_ref, dis_ref):
    deg = degp_ref[0] + degp_ref[1] + 1.0
    dis = lax.rsqrt(deg)
    xw = jnp.dot(x_ref[...], w_ref[...], preferred_element_type=jnp.float32)
    y_ref[...] = dis * xw
    dis_ref[...] = dis


def _tc_prep(degp, x, w, npad, d, blk):
    grid = (npad // blk,)
    return pl.pallas_call(
        _tc_prep_body,
        grid=grid,
        in_specs=[
            pl.BlockSpec((2, blk, 1), lambda i: (0, i, 0)),
            pl.BlockSpec((blk, d), lambda i: (i, 0)),
            pl.BlockSpec((d, d), lambda i: (0, 0)),
        ],
        out_specs=[
            pl.BlockSpec((blk, d), lambda i: (i, 0)),
            pl.BlockSpec((blk, 1), lambda i: (i, 0)),
        ],
        out_shape=[
            jax.ShapeDtypeStruct((npad, d), jnp.float32),
            jax.ShapeDtypeStruct((npad, 1), jnp.float32),
        ],
    )(degp, x, w)


def _tc_mid_body(dis_ref, p_ref, y_ref, w_ref, b_ref, y2_ref):
    dis = dis_ref[...]
    h = jnp.maximum(dis * (p_ref[0] + p_ref[1] + y_ref[...]) + b_ref[...], 0.0)
    y2_ref[...] = dis * jnp.dot(h, w_ref[...], preferred_element_type=jnp.float32)


def _tc_mid(dis, p, y, w, b, npad, d, blk):
    grid = (npad // blk,)
    return pl.pallas_call(
        _tc_mid_body,
        grid=grid,
        in_specs=[
            pl.BlockSpec((blk, 1), lambda i: (i, 0)),
            pl.BlockSpec((2, blk, d), lambda i: (0, i, 0)),
            pl.BlockSpec((blk, d), lambda i: (i, 0)),
            pl.BlockSpec((d, d), lambda i: (0, 0)),
            pl.BlockSpec((1, d), lambda i: (0, 0)),
        ],
        out_specs=pl.BlockSpec((blk, d), lambda i: (i, 0)),
        out_shape=jax.ShapeDtypeStruct((npad, d), jnp.float32),
    )(dis, p, y, w, b)


def _tc_final_body(dis_ref, p_ref, y_ref, b_ref, batch_ref, wfc_ref, bfc_ref,
                   out_ref, pooled, cnt, *, blk, nblocks):
    i = pl.program_id(0)

    @pl.when(i == 0)
    def _():
        pooled[...] = jnp.zeros_like(pooled)
        cnt[...] = jnp.zeros_like(cnt)

    dis = dis_ref[...]
    h = jnp.maximum(dis * (p_ref[0] + p_ref[1] + y_ref[...]) + b_ref[...], 0.0)
    labels = lax.broadcasted_iota(jnp.int32, (blk, NGRAPH), 1)
    onehot = (batch_ref[...] == labels).astype(jnp.float32)
    pooled[...] += lax.dot_general(
        onehot, h, (((0,), (0,)), ((), ())), preferred_element_type=jnp.float32)
    cnt[...] += jnp.sum(onehot, axis=0, keepdims=True)

    @pl.when(i == nblocks - 1)
    def _():
        g = pooled[...] / jnp.maximum(cnt[...], 1.0).T
        out_ref[...] = jnp.dot(g, wfc_ref[...],
                               preferred_element_type=jnp.float32) + bfc_ref[...]


def _tc_final(dis, p, y, b, batch2d, wfc, bfc, npad, d, o, blk):
    nblocks = npad // blk
    body = functools.partial(_tc_final_body, blk=blk, nblocks=nblocks)
    return pl.pallas_call(
        body,
        grid=(nblocks,),
        in_specs=[
            pl.BlockSpec((blk, 1), lambda i: (i, 0)),
            pl.BlockSpec((2, blk, d), lambda i: (0, i, 0)),
            pl.BlockSpec((blk, d), lambda i: (i, 0)),
            pl.BlockSpec((1, d), lambda i: (0, 0)),
            pl.BlockSpec((blk, 1), lambda i: (i, 0)),
            pl.BlockSpec((d, o), lambda i: (0, 0)),
            pl.BlockSpec((1, o), lambda i: (0, 0)),
        ],
        out_specs=pl.BlockSpec((NGRAPH, o), lambda i: (0, 0)),
        out_shape=jax.ShapeDtypeStruct((NGRAPH, o), jnp.float32),
        scratch_shapes=[
            pltpu.VMEM((NGRAPH, d), jnp.float32),
            pltpu.VMEM((1, NGRAPH), jnp.float32),
        ],
    )(dis, p, y, b, batch2d, wfc, bfc)


@jax.jit
def kernel(x, edge_index, batch, W1, b1, W2, b2, Wfc, bfc):
    n, d = x.shape
    e = edge_index.shape[1]
    o = Wfc.shape[1]
    blk = 1024

    npad = _round_up(n + 1, max(blk, NSUB * CHUNK))
    epad = _round_up(e, NW * CHUNK * 4)
    nch = epad // (NW * CHUNK)

    x_p = jnp.concatenate([x, jnp.zeros((npad - n, d), x.dtype)])
    batch_p = jnp.concatenate(
        [batch, jnp.full((npad - n,), NGRAPH, batch.dtype)]).reshape(npad, 1)
    pad_nodes = n + jnp.arange(epad - e, dtype=edge_index.dtype) % (npad - n)
    row_p = jnp.concatenate([edge_index[0], pad_nodes])
    col_p = jnp.concatenate([edge_index[1], pad_nodes])
    idx2 = (row_p | (col_p << 16)).reshape(NW, nch, 1, CHUNK)
    cidx = col_p.reshape(NW, nch, CHUNK)

    degp = _sc_degree(cidx, npad, nch).reshape(2, npad, 1)
    y1, dis = _tc_prep(degp, x_p, W1, npad, d, blk)
    p1 = _sc_agg(y1, idx2, npad, nch, d)
    y2 = _tc_mid(dis, p1, y1, W2, b1.reshape(1, d), npad, d, blk)
    p2 = _sc_agg(y2, idx2, npad, nch, d)
    return _tc_final(dis, p2, y2, b2.reshape(1, d), batch_p, Wfc,
                     bfc.reshape(1, o), npad, d, o, blk)


# speedup vs baseline: 1.0320x; 1.0316x over previous
import functools
import jax
import jax.numpy as jnp
from jax import lax
from jax.experimental import pallas as pl
from jax.experimental.pallas import tpu as pltpu
from jax.experimental.pallas import tpu_sc as plsc

NGRAPH = 32
NW = 32
NSUB = 16
CHUNK = 64
LANES = 16

_MESH = plsc.VectorSubcoreMesh(core_axis_name="c", subcore_axis_name="s")


def _round_up(a, m):
    return (a + m - 1) // m * m


def _sc_degree(cidx, npad, nch):
    stripe = npad // NSUB

    nslot = 4

    def body(cidx_hbm, deg_hbm, cidx_v, ones_v, zb_v, acc, *sems):
        c = lax.axis_index("c")
        s = lax.axis_index("s")
        wid = s * 2 + c
        for i in range(stripe // LANES):
            zb_v[pl.ds(i * LANES, LANES)] = jnp.zeros((LANES,), jnp.float32)
        for i in range(CHUNK // LANES):
            ones_v[pl.ds(i * LANES, LANES)] = jnp.ones((LANES,), jnp.float32)
        pltpu.sync_copy(zb_v, acc.at[pl.ds(s * stripe, stripe)])
        pltpu.sync_copy(cidx_hbm.at[wid], cidx_v)
        plsc.subcore_barrier()

        def chunk(i, carry):
            for b in range(nslot):
                k = nslot * i + b

                @pl.when(k >= nslot)
                def _(b=b, k=k):
                    pltpu.make_async_copy(
                        ones_v, acc.at[cidx_v.at[k - nslot]], sems[b]).wait()

                pltpu.async_copy(ones_v, acc.at[cidx_v.at[k]], sems[b],
                                 add=True)
            return carry

        lax.fori_loop(0, nch // nslot, chunk, 0)
        for b in range(nslot):
            pltpu.make_async_copy(
                ones_v, acc.at[cidx_v.at[nch - nslot + b]], sems[b]).wait()
        plsc.subcore_barrier()
        pltpu.sync_copy(acc.at[pl.ds(s * stripe, stripe)],
                        deg_hbm.at[c, pl.ds(s * stripe, stripe)])

    fn = pl.kernel(
        body,
        out_type=jax.ShapeDtypeStruct((2, npad), jnp.float32),
        mesh=_MESH,
        scratch_types=[
            pltpu.VMEM((nch, CHUNK), jnp.int32),
            pltpu.VMEM((CHUNK,), jnp.float32),
            pltpu.VMEM((stripe,), jnp.float32),
            pltpu.VMEM_SHARED((npad,), jnp.float32),
        ] + [pltpu.SemaphoreType.DMA] * nslot,
    )
    return fn(cidx)


def _sc_agg(y, idx2, npad, nch, d):
    stripe = npad // NSUB

    nslot = 4

    def body(y_hbm, idx_hbm, out_hbm,
             ib0, ib1, ib2, ib3, ixr0, ixr1, ixr2, ixr3,
             ixc0, ixc1, ixc2, ixc3, buf0, buf1, buf2, buf3, acc, *sem):
        c = lax.axis_index("c")
        s = lax.axis_index("s")
        wid = s * 2 + c
        ibs = (ib0, ib1, ib2, ib3)
        ixrs = (ixr0, ixr1, ixr2, ixr3)
        ixcs = (ixc0, ixc1, ixc2, ixc3)
        bufs = (buf0, buf1, buf2, buf3)
        semi, semg, sems = sem[:nslot], sem[nslot:2 * nslot], sem[2 * nslot:]

        for b in range(nslot):
            pltpu.async_copy(idx_hbm.at[wid, b], ibs[b], semi[b])

        zr = CHUNK
        def zrow(i, carry):
            for j in range(d // LANES):
                buf0[i, pl.ds(j * LANES, LANES)] = jnp.zeros((LANES,), jnp.float32)
            return carry

        lax.fori_loop(0, zr, zrow, 0)

        def zissue(i, carry):
            pltpu.async_copy(buf0.at[pl.ds(0, zr)],
                             acc.at[pl.ds(s * stripe + i * zr, zr)], sem[-1])
            return carry

        def zdrain(i, carry):
            pltpu.make_async_copy(
                buf0.at[pl.ds(0, zr)],
                acc.at[pl.ds(s * stripe + i * zr, zr)], sem[-1]).wait()
            return carry

        lax.fori_loop(0, stripe // zr, zissue, 0)
        lax.fori_loop(0, stripe // zr, zdrain, 0)
        plsc.subcore_barrier()

        def unpack(ib, ixr, ixc):
            for j in range(CHUNK // LANES):
                w = ib[0, pl.ds(j * LANES, LANES)]
                ixr[pl.ds(j * LANES, LANES)] = w & 0xFFFF
                ixc[pl.ds(j * LANES, LANES)] = lax.shift_right_logical(w, 16)

        def chunk(i, carry):
            for b in range(nslot):
                kb = nslot * i + b

                @pl.when(kb >= nslot)
                def _(b=b):
                    pltpu.make_async_copy(
                        bufs[b], acc.at[ixcs[b]], sems[b]).wait()

                pltpu.make_async_copy(
                    idx_hbm.at[wid, kb], ibs[b], semi[b]).wait()
                unpack(ibs[b], ixrs[b], ixcs[b])

                @pl.when(kb + nslot < nch)
                def _(b=b, kb=kb):
                    pltpu.async_copy(
                        idx_hbm.at[wid, kb + nslot], ibs[b], semi[b])

                pltpu.async_copy(y_hbm.at[ixrs[b]], bufs[b], semg[b])

            for b in range(nslot):
                pltpu.make_async_copy(
                    y_hbm.at[ixrs[b]], bufs[b], semg[b]).wait()
                pltpu.async_copy(
                    bufs[b], acc.at[ixcs[b]], sems[b], add=True)
            return carry

        lax.fori_loop(0, nch // nslot, chunk, 0)
        for b in range(nslot):
            pltpu.make_async_copy(bufs[b], acc.at[ixcs[b]],
                                  sems[b]).wait()
        plsc.subcore_barrier()
        pltpu.sync_copy(acc.at[pl.ds(s * stripe, stripe)],
                        out_hbm.at[c, pl.ds(s * stripe, stripe)])

    fn = pl.kernel(
        body,
        out_type=jax.ShapeDtypeStruct((2, npad, d), jnp.float32),
        mesh=_MESH,
        scratch_types=(
            [pltpu.VMEM((1, CHUNK), jnp.int32)] * nslot
            + [pltpu.VMEM((CHUNK,), jnp.int32)] * (2 * nslot)
            + [pltpu.VMEM((CHUNK, d), jnp.float32)] * nslot
            + [pltpu.VMEM_SHARED((npad, d), jnp.float32)]
            + [pltpu.SemaphoreType.DMA] * (3 * nslot + 1)
        ),
    )
    return fn(y, idx2)


def _tc_prep_body(degp_ref, x_ref, w_ref, y_ref, dis_ref):
    deg = degp_ref[0] + degp_ref[1] + 1.0
    dis = lax.rsqrt(deg)
    xw = jnp.dot(x_ref[...], w_ref[...], preferred_element_type=jnp.float32)
    y_ref[...] = dis * xw
    dis_ref[...] = dis


def _tc_prep(degp, x, w, npad, d, blk):
    grid = (npad // blk,)
    return pl.pallas_call(
        _tc_prep_body,
        grid=grid,
        in_specs=[
            pl.BlockSpec((2, blk, 1), lambda i: (0, i, 0)),
            pl.BlockSpec((blk, d), lambda i: (i, 0)),
            pl.BlockSpec((d, d), lambda i: (0, 0)),
        ],
        out_specs=[
            pl.BlockSpec((blk, d), lambda i: (i, 0)),
            pl.BlockSpec((blk, 1), lambda i: (i, 0)),
        ],
        out_shape=[
            jax.ShapeDtypeStruct((npad, d), jnp.float32),
            jax.ShapeDtypeStruct((npad, 1), jnp.float32),
        ],
    )(degp, x, w)


def _tc_mid_body(dis_ref, p_ref, y_ref, w_ref, b_ref, y2_ref):
    dis = dis_ref[...]
    h = jnp.maximum(dis * (p_ref[0] + p_ref[1] + y_ref[...]) + b_ref[...], 0.0)
    y2_ref[...] = dis * jnp.dot(h, w_ref[...], preferred_element_type=jnp.float32)


def _tc_mid(dis, p, y, w, b, npad, d, blk):
    grid = (npad // blk,)
    return pl.pallas_call(
        _tc_mid_body,
        grid=grid,
        in_specs=[
            pl.BlockSpec((blk, 1), lambda i: (i, 0)),
            pl.BlockSpec((2, blk, d), lambda i: (0, i, 0)),
            pl.BlockSpec((blk, d), lambda i: (i, 0)),
            pl.BlockSpec((d, d), lambda i: (0, 0)),
            pl.BlockSpec((1, d), lambda i: (0, 0)),
        ],
        out_specs=pl.BlockSpec((blk, d), lambda i: (i, 0)),
        out_shape=jax.ShapeDtypeStruct((npad, d), jnp.float32),
    )(dis, p, y, w, b)


def _tc_final_body(dis_ref, p_ref, y_ref, b_ref, batch_ref, wfc_ref, bfc_ref,
                   out_ref, pooled, cnt, *, blk, nblocks):
    i = pl.program_id(0)

    @pl.when(i == 0)
    def _():
        pooled[...] = jnp.zeros_like(pooled)
        cnt[...] = jnp.zeros_like(cnt)

    dis = dis_ref[...]
    h = jnp.maximum(dis * (p_ref[0] + p_ref[1] + y_ref[...]) + b_ref[...], 0.0)
    labels = lax.broadcasted_iota(jnp.int32, (blk, NGRAPH), 1)
    onehot = (batch_ref[...] == labels).astype(jnp.float32)
    pooled[...] += lax.dot_general(
        onehot, h, (((0,), (0,)), ((), ())), preferred_element_type=jnp.float32)
    cnt[...] += jnp.sum(onehot, axis=0, keepdims=True)

    @pl.when(i == nblocks - 1)
    def _():
        g = pooled[...] / jnp.maximum(cnt[...], 1.0).T
        out_ref[...] = jnp.dot(g, wfc_ref[...],
                               preferred_element_type=jnp.float32) + bfc_ref[...]


def _tc_final(dis, p, y, b, batch2d, wfc, bfc, npad, d, o, blk):
    nblocks = npad // blk
    body = functools.partial(_tc_final_body, blk=blk, nblocks=nblocks)
    return pl.pallas_call(
        body,
        grid=(nblocks,),
        in_specs=[
            pl.BlockSpec((blk, 1), lambda i: (i, 0)),
            pl.BlockSpec((2, blk, d), lambda i: (0, i, 0)),
            pl.BlockSpec((blk, d), lambda i: (i, 0)),
            pl.BlockSpec((1, d), lambda i: (0, 0)),
            pl.BlockSpec((blk, 1), lambda i: (i, 0)),
            pl.BlockSpec((d, o), lambda i: (0, 0)),
            pl.BlockSpec((1, o), lambda i: (0, 0)),
        ],
        out_specs=pl.BlockSpec((NGRAPH, o), lambda i: (0, 0)),
        out_shape=jax.ShapeDtypeStruct((NGRAPH, o), jnp.float32),
        scratch_shapes=[
            pltpu.VMEM((NGRAPH, d), jnp.float32),
            pltpu.VMEM((1, NGRAPH), jnp.float32),
        ],
    )(dis, p, y, b, batch2d, wfc, bfc)


@jax.jit
def kernel(x, edge_index, batch, W1, b1, W2, b2, Wfc, bfc):
    n, d = x.shape
    e = edge_index.shape[1]
    o = Wfc.shape[1]
    blk = 1024

    npad = _round_up(n + 1, max(blk, NSUB * CHUNK))
    epad = _round_up(e, NW * CHUNK * 4)
    nch = epad // (NW * CHUNK)

    x_p = jnp.concatenate([x, jnp.zeros((npad - n, d), x.dtype)])
    batch_p = jnp.concatenate(
        [batch, jnp.full((npad - n,), NGRAPH, batch.dtype)]).reshape(npad, 1)
    pad_nodes = n + jnp.arange(epad - e, dtype=edge_index.dtype) % (npad - n)
    row_p = jnp.concatenate([edge_index[0], pad_nodes])
    col_p = jnp.concatenate([edge_index[1], pad_nodes])
    idx2 = (row_p | (col_p << 16)).reshape(NW, nch, 1, CHUNK)
    cidx = col_p.reshape(NW, nch, CHUNK)

    degp = _sc_degree(cidx, npad, nch).reshape(2, npad, 1)
    y1, dis = _tc_prep(degp, x_p, W1, npad, d, blk)
    p1 = _sc_agg(y1, idx2, npad, nch, d)
    y2 = _tc_mid(dis, p1, y1, W2, b1.reshape(1, d), npad, d, blk)
    p2 = _sc_agg(y2, idx2, npad, nch, d)
    return _tc_final(dis, p2, y2, b2.reshape(1, d), batch_p, Wfc,
                     bfc.reshape(1, o), npad, d, o, blk)


# speedup vs baseline: 1.0517x; 1.0190x over previous
import functools
import jax
import jax.numpy as jnp
from jax import lax
from jax.experimental import pallas as pl
from jax.experimental.pallas import tpu as pltpu
from jax.experimental.pallas import tpu_sc as plsc

NGRAPH = 32
NW = 32
NSUB = 16
CHUNK = 64
LANES = 16

_MESH = plsc.VectorSubcoreMesh(core_axis_name="c", subcore_axis_name="s")


def _round_up(a, m):
    return (a + m - 1) // m * m


def _sc_degree(cidx, npad, nch):
    stripe = npad // NSUB

    nslot = 4

    def body(cidx_hbm, deg_hbm, cidx_v, ones_v, zb_v, acc, *sems):
        c = lax.axis_index("c")
        s = lax.axis_index("s")
        wid = s * 2 + c
        for i in range(stripe // LANES):
            zb_v[pl.ds(i * LANES, LANES)] = jnp.zeros((LANES,), jnp.float32)
        for i in range(CHUNK // LANES):
            ones_v[pl.ds(i * LANES, LANES)] = jnp.ones((LANES,), jnp.float32)
        pltpu.sync_copy(zb_v, acc.at[pl.ds(s * stripe, stripe)])
        pltpu.sync_copy(cidx_hbm.at[wid], cidx_v)
        plsc.subcore_barrier()

        def chunk(i, carry):
            for b in range(nslot):
                k = nslot * i + b

                @pl.when(k >= nslot)
                def _(b=b, k=k):
                    pltpu.make_async_copy(
                        ones_v, acc.at[cidx_v.at[k - nslot]], sems[b]).wait()

                pltpu.async_copy(ones_v, acc.at[cidx_v.at[k]], sems[b],
                                 add=True)
            return carry

        lax.fori_loop(0, nch // nslot, chunk, 0)
        for b in range(nslot):
            pltpu.make_async_copy(
                ones_v, acc.at[cidx_v.at[nch - nslot + b]], sems[b]).wait()
        plsc.subcore_barrier()
        pltpu.sync_copy(acc.at[pl.ds(s * stripe, stripe)],
                        deg_hbm.at[c, pl.ds(s * stripe, stripe)])

    fn = pl.kernel(
        body,
        out_type=jax.ShapeDtypeStruct((2, npad), jnp.float32),
        mesh=_MESH,
        scratch_types=[
            pltpu.VMEM((nch, CHUNK), jnp.int32),
            pltpu.VMEM((CHUNK,), jnp.float32),
            pltpu.VMEM((stripe,), jnp.float32),
            pltpu.VMEM_SHARED((npad,), jnp.float32),
        ] + [pltpu.SemaphoreType.DMA] * nslot,
    )
    return fn(cidx)


def _sc_agg(y, idx2, npad, nch, d):
    stripe = npad // NSUB

    nslot = 4

    def body(y_hbm, idx_hbm, out_hbm,
             ib0, ib1, ib2, ib3, ixr0, ixr1, ixr2, ixr3,
             ixc0, ixc1, ixc2, ixc3, buf0, buf1, buf2, buf3, acc, *sem):
        c = lax.axis_index("c")
        s = lax.axis_index("s")
        wid = s * 2 + c
        ibs = (ib0, ib1, ib2, ib3)
        ixrs = (ixr0, ixr1, ixr2, ixr3)
        ixcs = (ixc0, ixc1, ixc2, ixc3)
        bufs = (buf0, buf1, buf2, buf3)
        semi, semg, sems = sem[:nslot], sem[nslot:2 * nslot], sem[2 * nslot:]

        for b in range(nslot):
            pltpu.async_copy(idx_hbm.at[wid, b], ibs[b], semi[b])

        zr = CHUNK
        def zrow(i, carry):
            for j in range(d // LANES):
                buf0[i, pl.ds(j * LANES, LANES)] = jnp.zeros((LANES,), jnp.float32)
            return carry

        lax.fori_loop(0, zr, zrow, 0)

        def zissue(i, carry):
            pltpu.async_copy(buf0.at[pl.ds(0, zr)],
                             acc.at[pl.ds(s * stripe + i * zr, zr)], sem[-1])
            return carry

        def zdrain(i, carry):
            pltpu.make_async_copy(
                buf0.at[pl.ds(0, zr)],
                acc.at[pl.ds(s * stripe + i * zr, zr)], sem[-1]).wait()
            return carry

        lax.fori_loop(0, stripe // zr, zissue, 0)
        lax.fori_loop(0, stripe // zr, zdrain, 0)
        plsc.subcore_barrier()

        def unpack(ib, ixr, ixc):
            for j in range(CHUNK // LANES):
                w = ib[0, pl.ds(j * LANES, LANES)]
                ixr[pl.ds(j * LANES, LANES)] = w & 0xFFFF
                ixc[pl.ds(j * LANES, LANES)] = lax.shift_right_logical(w, 16)

        def chunk(i, carry):
            for b in range(nslot):
                kb = nslot * i + b

                @pl.when(kb >= nslot)
                def _(b=b):
                    pltpu.make_async_copy(
                        bufs[b], acc.at[ixcs[b]], sems[b]).wait()

                pltpu.make_async_copy(
                    idx_hbm.at[wid, kb], ibs[b], semi[b]).wait()
                unpack(ibs[b], ixrs[b], ixcs[b])

                @pl.when(kb + nslot < nch)
                def _(b=b, kb=kb):
                    pltpu.async_copy(
                        idx_hbm.at[wid, kb + nslot], ibs[b], semi[b])

                pltpu.async_copy(y_hbm.at[ixrs[b]], bufs[b], semg[b])

            for b in range(nslot):
                pltpu.make_async_copy(
                    y_hbm.at[ixrs[b]], bufs[b], semg[b]).wait()
                pltpu.async_copy(
                    bufs[b], acc.at[ixcs[b]], sems[b], add=True)
            return carry

        lax.fori_loop(0, nch // nslot, chunk, 0)
        for b in range(nslot):
            pltpu.make_async_copy(bufs[b], acc.at[ixcs[b]],
                                  sems[b]).wait()
        plsc.subcore_barrier()
        pltpu.sync_copy(acc.at[pl.ds(s * stripe, stripe)],
                        out_hbm.at[c, pl.ds(s * stripe, stripe)])

    fn = pl.kernel(
        body,
        out_type=jax.ShapeDtypeStruct((2, npad, d), jnp.float32),
        mesh=_MESH,
        scratch_types=(
            [pltpu.VMEM((1, CHUNK), jnp.int32)] * nslot
            + [pltpu.VMEM((CHUNK,), jnp.int32)] * (2 * nslot)
            + [pltpu.VMEM((CHUNK, d), jnp.float32)] * nslot
            + [pltpu.VMEM_SHARED((npad, d), jnp.float32)]
            + [pltpu.SemaphoreType.DMA] * (3 * nslot + 1)
        ),
    )
    return fn(y, idx2)


def _tc_prep_body(degp_ref, x_ref, w_ref, y_ref, dis_ref):
    deg = degp_ref[0] + degp_ref[1] + 1.0
    dis = lax.rsqrt(deg)
    xw = jnp.dot(x_ref[...], w_ref[...], preferred_element_type=jnp.float32)
    y_ref[...] = dis * xw
    dis_ref[...] = dis


def _tc_prep(degp, x, w, npad, d, blk):
    grid = (npad // blk,)
    return pl.pallas_call(
        _tc_prep_body,
        grid=grid,
        in_specs=[
            pl.BlockSpec((2, blk, 1), lambda i: (0, i, 0)),
            pl.BlockSpec((blk, d), lambda i: (i, 0)),
            pl.BlockSpec((d, d), lambda i: (0, 0)),
        ],
        out_specs=[
            pl.BlockSpec((blk, d), lambda i: (i, 0)),
            pl.BlockSpec((blk, 1), lambda i: (i, 0)),
        ],
        out_shape=[
            jax.ShapeDtypeStruct((npad, d), jnp.float32),
            jax.ShapeDtypeStruct((npad, 1), jnp.float32),
        ],
    )(degp, x, w)


def _tc_mid_body(dis_ref, p_ref, y_ref, w_ref, b_ref, y2_ref):
    dis = dis_ref[...]
    h = jnp.maximum(dis * (p_ref[0] + p_ref[1] + y_ref[...]) + b_ref[...], 0.0)
    y2_ref[...] = dis * jnp.dot(h, w_ref[...], preferred_element_type=jnp.float32)


def _tc_mid(dis, p, y, w, b, npad, d, blk):
    grid = (npad // blk,)
    return pl.pallas_call(
        _tc_mid_body,
        grid=grid,
        in_specs=[
            pl.BlockSpec((blk, 1), lambda i: (i, 0)),
            pl.BlockSpec((2, blk, d), lambda i: (0, i, 0)),
            pl.BlockSpec((blk, d), lambda i: (i, 0)),
            pl.BlockSpec((d, d), lambda i: (0, 0)),
            pl.BlockSpec((1, d), lambda i: (0, 0)),
        ],
        out_specs=pl.BlockSpec((blk, d), lambda i: (i, 0)),
        out_shape=jax.ShapeDtypeStruct((npad, d), jnp.float32),
    )(dis, p, y, w, b)


def _tc_final_body(dis_ref, p_ref, y_ref, b_ref, batch_ref, wfc_ref, bfc_ref,
                   out_ref, pooled, cnt, *, blk, nblocks):
    i = pl.program_id(0)

    @pl.when(i == 0)
    def _():
        pooled[...] = jnp.zeros_like(pooled)
        cnt[...] = jnp.zeros_like(cnt)

    dis = dis_ref[...]
    h = jnp.maximum(dis * (p_ref[0] + p_ref[1] + y_ref[...]) + b_ref[...], 0.0)
    labels = lax.broadcasted_iota(jnp.int32, (blk, NGRAPH), 1)
    onehot = (batch_ref[...] == labels).astype(jnp.float32)
    pooled[...] += lax.dot_general(
        onehot, h, (((0,), (0,)), ((), ())), preferred_element_type=jnp.float32)
    cnt[...] += jnp.sum(onehot, axis=0, keepdims=True)

    @pl.when(i == nblocks - 1)
    def _():
        g = pooled[...] / jnp.maximum(cnt[...], 1.0).T
        out_ref[...] = jnp.dot(g, wfc_ref[...],
                               preferred_element_type=jnp.float32) + bfc_ref[...]


def _tc_final(dis, p, y, b, batch2d, wfc, bfc, npad, d, o, blk):
    nblocks = npad // blk
    body = functools.partial(_tc_final_body, blk=blk, nblocks=nblocks)
    return pl.pallas_call(
        body,
        grid=(nblocks,),
        in_specs=[
            pl.BlockSpec((blk, 1), lambda i: (i, 0)),
            pl.BlockSpec((2, blk, d), lambda i: (0, i, 0)),
            pl.BlockSpec((blk, d), lambda i: (i, 0)),
            pl.BlockSpec((1, d), lambda i: (0, 0)),
            pl.BlockSpec((blk, 1), lambda i: (i, 0)),
            pl.BlockSpec((d, o), lambda i: (0, 0)),
            pl.BlockSpec((1, o), lambda i: (0, 0)),
        ],
        out_specs=pl.BlockSpec((NGRAPH, o), lambda i: (0, 0)),
        out_shape=jax.ShapeDtypeStruct((NGRAPH, o), jnp.float32),
        scratch_shapes=[
            pltpu.VMEM((NGRAPH, d), jnp.float32),
            pltpu.VMEM((1, NGRAPH), jnp.float32),
        ],
    )(dis, p, y, b, batch2d, wfc, bfc)


@jax.jit
def kernel(x, edge_index, batch, W1, b1, W2, b2, Wfc, bfc):
    n, d = x.shape
    e = edge_index.shape[1]
    o = Wfc.shape[1]

    npad = _round_up(n + 1, NSUB * CHUNK)
    blk = 2048
    while npad % blk:
        blk //= 2
    epad = _round_up(e, NW * CHUNK * 4)
    nch = epad // (NW * CHUNK)

    x_p = jnp.concatenate([x, jnp.zeros((npad - n, d), x.dtype)])
    batch_p = jnp.concatenate(
        [batch, jnp.full((npad - n,), NGRAPH, batch.dtype)]).reshape(npad, 1)
    pad_nodes = n + jnp.arange(epad - e, dtype=edge_index.dtype) % (npad - n)
    row_p = jnp.concatenate([edge_index[0], pad_nodes])
    col_p = jnp.concatenate([edge_index[1], pad_nodes])
    idx2 = (row_p | (col_p << 16)).reshape(NW, nch, 1, CHUNK)
    cidx = col_p.reshape(NW, nch, CHUNK)

    degp = _sc_degree(cidx, npad, nch).reshape(2, npad, 1)
    y1, dis = _tc_prep(degp, x_p, W1, npad, d, blk)
    p1 = _sc_agg(y1, idx2, npad, nch, d)
    y2 = _tc_mid(dis, p1, y1, W2, b1.reshape(1, d), npad, d, blk)
    p2 = _sc_agg(y2, idx2, npad, nch, d)
    return _tc_final(dis, p2, y2, b2.reshape(1, d), batch_p, Wfc,
                     bfc.reshape(1, o), npad, d, o, blk)


# speedup vs baseline: 1.0580x; 1.0060x over previous
import functools
import jax
import jax.numpy as jnp
from jax import lax
from jax.experimental import pallas as pl
from jax.experimental.pallas import tpu as pltpu
from jax.experimental.pallas import tpu_sc as plsc

NGRAPH = 32
NW = 32
NSUB = 16
CHUNK = 64
LANES = 16

_MESH = plsc.VectorSubcoreMesh(core_axis_name="c", subcore_axis_name="s")


def _round_up(a, m):
    return (a + m - 1) // m * m


def _sc_degree(cidx, npad, nch):
    stripe = npad // NSUB

    nslot = 4

    def body(cidx_hbm, deg_hbm, cidx_v, ones_v, zb_v, acc, *sems):
        c = lax.axis_index("c")
        s = lax.axis_index("s")
        wid = s * 2 + c
        for i in range(stripe // LANES):
            zb_v[pl.ds(i * LANES, LANES)] = jnp.zeros((LANES,), jnp.float32)
        for i in range(CHUNK // LANES):
            ones_v[pl.ds(i * LANES, LANES)] = jnp.ones((LANES,), jnp.float32)
        pltpu.sync_copy(zb_v, acc.at[pl.ds(s * stripe, stripe)])
        pltpu.sync_copy(cidx_hbm.at[wid], cidx_v)
        plsc.subcore_barrier()

        def chunk(i, carry):
            for b in range(nslot):
                k = nslot * i + b

                @pl.when(k >= nslot)
                def _(b=b, k=k):
                    pltpu.make_async_copy(
                        ones_v, acc.at[cidx_v.at[k - nslot]], sems[b]).wait()

                pltpu.async_copy(ones_v, acc.at[cidx_v.at[k]], sems[b],
                                 add=True)
            return carry

        lax.fori_loop(0, nch // nslot, chunk, 0)
        for b in range(nslot):
            pltpu.make_async_copy(
                ones_v, acc.at[cidx_v.at[nch - nslot + b]], sems[b]).wait()
        plsc.subcore_barrier()
        pltpu.sync_copy(acc.at[pl.ds(s * stripe, stripe)],
                        deg_hbm.at[c, pl.ds(s * stripe, stripe)])

    fn = pl.kernel(
        body,
        out_type=jax.ShapeDtypeStruct((2, npad), jnp.float32),
        mesh=_MESH,
        scratch_types=[
            pltpu.VMEM((nch, CHUNK), jnp.int32),
            pltpu.VMEM((CHUNK,), jnp.float32),
            pltpu.VMEM((stripe,), jnp.float32),
            pltpu.VMEM_SHARED((npad,), jnp.float32),
        ] + [pltpu.SemaphoreType.DMA] * nslot,
    )
    return fn(cidx)


def _sc_agg(y, idx2, npad, nch, d):
    stripe = npad // NSUB

    nslot = 4

    def body(y_hbm, idx_hbm, out_hbm,
             ib0, ib1, ib2, ib3, ixr0, ixr1, ixr2, ixr3,
             ixc0, ixc1, ixc2, ixc3, buf0, buf1, buf2, buf3, acc, *sem):
        c = lax.axis_index("c")
        s = lax.axis_index("s")
        wid = s * 2 + c
        ibs = (ib0, ib1, ib2, ib3)
        ixrs = (ixr0, ixr1, ixr2, ixr3)
        ixcs = (ixc0, ixc1, ixc2, ixc3)
        bufs = (buf0, buf1, buf2, buf3)
        semi, semg, sems = sem[:nslot], sem[nslot:2 * nslot], sem[2 * nslot:]

        for b in range(nslot):
            pltpu.async_copy(idx_hbm.at[wid, b], ibs[b], semi[b])

        zr = CHUNK
        def zrow(i, carry):
            for j in range(d // LANES):
                buf0[i, pl.ds(j * LANES, LANES)] = jnp.zeros((LANES,), jnp.float32)
            return carry

        lax.fori_loop(0, zr, zrow, 0)

        def zissue(i, carry):
            pltpu.async_copy(buf0.at[pl.ds(0, zr)],
                             acc.at[pl.ds(s * stripe + i * zr, zr)], sem[-1])
            return carry

        def zdrain(i, carry):
            pltpu.make_async_copy(
                buf0.at[pl.ds(0, zr)],
                acc.at[pl.ds(s * stripe + i * zr, zr)], sem[-1]).wait()
            return carry

        lax.fori_loop(0, stripe // zr, zissue, 0)
        lax.fori_loop(0, stripe // zr, zdrain, 0)
        plsc.subcore_barrier()

        def unpack(ib, ixr, ixc):
            for j in range(CHUNK // LANES):
                w = ib[0, pl.ds(j * LANES, LANES)]
                ixr[pl.ds(j * LANES, LANES)] = w & 0xFFFF
                ixc[pl.ds(j * LANES, LANES)] = lax.shift_right_logical(w, 16)

        def chunk(i, carry):
            for b in range(nslot):
                kb = nslot * i + b

                @pl.when(kb >= nslot)
                def _(b=b):
                    pltpu.make_async_copy(
                        bufs[b], acc.at[ixcs[b]], sems[b]).wait()

                pltpu.make_async_copy(
                    idx_hbm.at[wid, kb], ibs[b], semi[b]).wait()
                unpack(ibs[b], ixrs[b], ixcs[b])

                @pl.when(kb + nslot < nch)
                def _(b=b, kb=kb):
                    pltpu.async_copy(
                        idx_hbm.at[wid, kb + nslot], ibs[b], semi[b])

                pltpu.async_copy(y_hbm.at[ixrs[b]], bufs[b], semg[b])

            for b in range(nslot):
                pltpu.make_async_copy(
                    y_hbm.at[ixrs[b]], bufs[b], semg[b]).wait()
                pltpu.async_copy(
                    bufs[b], acc.at[ixcs[b]], sems[b], add=True)
            return carry

        lax.fori_loop(0, nch // nslot, chunk, 0)
        for b in range(nslot):
            pltpu.make_async_copy(bufs[b], acc.at[ixcs[b]],
                                  sems[b]).wait()
        plsc.subcore_barrier()
        pltpu.sync_copy(acc.at[pl.ds(s * stripe, stripe)],
                        out_hbm.at[c, pl.ds(s * stripe, stripe)])

    fn = pl.kernel(
        body,
        out_type=jax.ShapeDtypeStruct((2, npad, d), jnp.float32),
        mesh=_MESH,
        scratch_types=(
            [pltpu.VMEM((1, CHUNK), jnp.int32)] * nslot
            + [pltpu.VMEM((CHUNK,), jnp.int32)] * (2 * nslot)
            + [pltpu.VMEM((CHUNK, d), jnp.float32)] * nslot
            + [pltpu.VMEM_SHARED((npad, d), jnp.float32)]
            + [pltpu.SemaphoreType.DMA] * (3 * nslot + 1)
        ),
    )
    return fn(y, idx2)


def _tc_prep_body(degp_ref, x_ref, w_ref, y_ref, dis_ref):
    deg = degp_ref[0] + degp_ref[1] + 1.0
    dis = lax.rsqrt(deg)
    xw = jnp.dot(x_ref[...], w_ref[...], preferred_element_type=jnp.float32)
    y_ref[...] = dis * xw
    dis_ref[...] = dis


def _tc_prep(degp, x, w, npad, d, blk):
    grid = (npad // blk,)
    return pl.pallas_call(
        _tc_prep_body,
        grid=grid,
        in_specs=[
            pl.BlockSpec((2, blk, 1), lambda i: (0, i, 0)),
            pl.BlockSpec((blk, d), lambda i: (i, 0)),
            pl.BlockSpec((d, d), lambda i: (0, 0)),
        ],
        out_specs=[
            pl.BlockSpec((blk, d), lambda i: (i, 0)),
            pl.BlockSpec((blk, 1), lambda i: (i, 0)),
        ],
        out_shape=[
            jax.ShapeDtypeStruct((npad, d), jnp.float32),
            jax.ShapeDtypeStruct((npad, 1), jnp.float32),
        ],
    )(degp, x, w)


def _tc_mid_body(dis_ref, p_ref, y_ref, w_ref, b_ref, y2_ref):
    dis = dis_ref[...]
    h = jnp.maximum(dis * (p_ref[0] + p_ref[1] + y_ref[...]) + b_ref[...], 0.0)
    y2_ref[...] = dis * jnp.dot(h, w_ref[...], preferred_element_type=jnp.float32)


def _tc_mid(dis, p, y, w, b, npad, d, blk):
    grid = (npad // blk,)
    return pl.pallas_call(
        _tc_mid_body,
        grid=grid,
        in_specs=[
            pl.BlockSpec((blk, 1), lambda i: (i, 0)),
            pl.BlockSpec((2, blk, d), lambda i: (0, i, 0)),
            pl.BlockSpec((blk, d), lambda i: (i, 0)),
            pl.BlockSpec((d, d), lambda i: (0, 0)),
            pl.BlockSpec((1, d), lambda i: (0, 0)),
        ],
        out_specs=pl.BlockSpec((blk, d), lambda i: (i, 0)),
        out_shape=jax.ShapeDtypeStruct((npad, d), jnp.float32),
    )(dis, p, y, w, b)


def _tc_final_body(dis_ref, p_ref, y_ref, b_ref, batch_ref, wfc_ref, bfc_ref,
                   out_ref, pooled, cnt, *, blk, nblocks):
    i = pl.program_id(0)

    @pl.when(i == 0)
    def _():
        pooled[...] = jnp.zeros_like(pooled)
        cnt[...] = jnp.zeros_like(cnt)

    dis = dis_ref[...]
    h = jnp.maximum(dis * (p_ref[0] + p_ref[1] + y_ref[...]) + b_ref[...], 0.0)
    labels = lax.broadcasted_iota(jnp.int32, (blk, NGRAPH), 1)
    onehot = (batch_ref[...] == labels).astype(jnp.float32)
    pooled[...] += lax.dot_general(
        onehot, h, (((0,), (0,)), ((), ())), preferred_element_type=jnp.float32)
    cnt[...] += jnp.sum(onehot, axis=0, keepdims=True)

    @pl.when(i == nblocks - 1)
    def _():
        g = pooled[...] / jnp.maximum(cnt[...], 1.0).T
        out_ref[...] = jnp.dot(g, wfc_ref[...],
                               preferred_element_type=jnp.float32) + bfc_ref[...]


def _tc_final(dis, p, y, b, batch2d, wfc, bfc, npad, d, o, blk):
    nblocks = npad // blk
    body = functools.partial(_tc_final_body, blk=blk, nblocks=nblocks)
    return pl.pallas_call(
        body,
        grid=(nblocks,),
        in_specs=[
            pl.BlockSpec((blk, 1), lambda i: (i, 0)),
            pl.BlockSpec((2, blk, d), lambda i: (0, i, 0)),
            pl.BlockSpec((blk, d), lambda i: (i, 0)),
            pl.BlockSpec((1, d), lambda i: (0, 0)),
            pl.BlockSpec((blk, 1), lambda i: (i, 0)),
            pl.BlockSpec((d, o), lambda i: (0, 0)),
            pl.BlockSpec((1, o), lambda i: (0, 0)),
        ],
        out_specs=pl.BlockSpec((NGRAPH, o), lambda i: (0, 0)),
        out_shape=jax.ShapeDtypeStruct((NGRAPH, o), jnp.float32),
        scratch_shapes=[
            pltpu.VMEM((NGRAPH, d), jnp.float32),
            pltpu.VMEM((1, NGRAPH), jnp.float32),
        ],
    )(dis, p, y, b, batch2d, wfc, bfc)


@jax.jit
def kernel(x, edge_index, batch, W1, b1, W2, b2, Wfc, bfc):
    n, d = x.shape
    e = edge_index.shape[1]
    o = Wfc.shape[1]

    npad = _round_up(n + 1, NSUB * CHUNK)
    blk = npad // 2 if npad % 2 == 0 and npad // 2 <= 5120 else 2048
    while npad % blk:
        blk //= 2
    epad = _round_up(e, NW * CHUNK * 4)
    nch = epad // (NW * CHUNK)

    x_p = jnp.concatenate([x, jnp.zeros((npad - n, d), x.dtype)])
    batch_p = jnp.concatenate(
        [batch, jnp.full((npad - n,), NGRAPH, batch.dtype)]).reshape(npad, 1)
    pad_nodes = n + jnp.arange(epad - e, dtype=edge_index.dtype) % (npad - n)
    row_p = jnp.concatenate([edge_index[0], pad_nodes])
    col_p = jnp.concatenate([edge_index[1], pad_nodes])
    idx2 = (row_p | (col_p << 16)).reshape(NW, nch, 1, CHUNK)
    cidx = col_p.reshape(NW, nch, CHUNK)

    degp = _sc_degree(cidx, npad, nch).reshape(2, npad, 1)
    y1, dis = _tc_prep(degp, x_p, W1, npad, d, blk)
    p1 = _sc_agg(y1, idx2, npad, nch, d)
    y2 = _tc_mid(dis, p1, y1, W2, b1.reshape(1, d), npad, d, blk)
    p2 = _sc_agg(y2, idx2, npad, nch, d)
    return _tc_final(dis, p2, y2, b2.reshape(1, d), batch_p, Wfc,
                     bfc.reshape(1, o), npad, d, o, blk)
